# baseline (speedup 1.0000x reference)
import jax
import jax.numpy as jnp
from jax.experimental import pallas as pl
from jax.experimental.pallas import tpu as pltpu


def _body(l_ref, f_ref, fm_ref, w1t_ref, w2t_ref, b_ref, out_ref):
    x = jnp.dot(
        l_ref[...].astype(jnp.bfloat16),
        f_ref[...].astype(jnp.bfloat16),
        preferred_element_type=jnp.float32,
    )
    f = fm_ref[...]
    out_ref[...] = (
        jnp.dot(f + x, w1t_ref[...], preferred_element_type=jnp.float32)
        + jnp.dot(x * f, w2t_ref[...], preferred_element_type=jnp.float32)
        + b_ref[...]
    )


def kernel(lap_matrix, eye_matrix, features, W1, b1, W2, b2):
    n, d = features.shape
    bm = min(128, n)
    nm = n // bm

    bias = (b1 + b2).reshape(1, d)

    in_specs = [
        pl.BlockSpec((bm, n), lambda i: (i, 0)),
        pl.BlockSpec((n, d), lambda i: (0, 0)),
        pl.BlockSpec((bm, d), lambda i: (i, 0)),
        pl.BlockSpec((d, d), lambda i: (0, 0)),
        pl.BlockSpec((d, d), lambda i: (0, 0)),
        pl.BlockSpec((1, d), lambda i: (0, 0)),
    ]

    return pl.pallas_call(
        _body,
        grid=(nm,),
        in_specs=in_specs,
        out_specs=pl.BlockSpec((bm, d), lambda i: (i, 0)),
        out_shape=jax.ShapeDtypeStruct((n, d), jnp.float32),
        compiler_params=pltpu.CompilerParams(
            dimension_semantics=("arbitrary",),
        ),
    )(lap_matrix, features, features, W1.T, W2.T, bias)


# speedup vs baseline: 1.0306x; 1.0306x over previous
import functools

import jax
import jax.numpy as jnp
from jax.experimental import pallas as pl
from jax.experimental.pallas import tpu as pltpu

_SPLIT = 2


def _body(sub, *refs):
    l_refs = refs[:_SPLIT]
    f_ref, fm_ref, w1t_ref, w2t_ref, b_ref, out_ref = refs[_SPLIT:]
    fb = f_ref[...].astype(jnp.bfloat16)
    w1t = w1t_ref[...]
    w2t = w2t_ref[...]
    b = b_ref[...]
    for j in range(_SPLIT):
        x = jnp.dot(
            l_refs[j][...].astype(jnp.bfloat16),
            fb,
            preferred_element_type=jnp.float32,
        )
        f = fm_ref[pl.ds(j * sub, sub), :]
        out_ref[pl.ds(j * sub, sub), :] = (
            jnp.dot(f + x, w1t, preferred_element_type=jnp.float32)
            + jnp.dot(x * f, w2t, preferred_element_type=jnp.float32)
            + b
        )


def kernel(lap_matrix, eye_matrix, features, W1, b1, W2, b2):
    n, d = features.shape
    bm = min(256, n)
    sub = bm // _SPLIT
    nm = n // bm

    bias = (b1 + b2).reshape(1, d)

    l_specs = [
        pl.BlockSpec((sub, n), functools.partial(lambda j, i: (_SPLIT * i + j, 0), j))
        for j in range(_SPLIT)
    ]
    in_specs = l_specs + [
        pl.BlockSpec((n, d), lambda i: (0, 0)),
        pl.BlockSpec((bm, d), lambda i: (i, 0)),
        pl.BlockSpec((d, d), lambda i: (0, 0)),
        pl.BlockSpec((d, d), lambda i: (0, 0)),
        pl.BlockSpec((1, d), lambda i: (0, 0)),
    ]

    return pl.pallas_call(
        functools.partial(_body, sub),
        grid=(nm,),
        in_specs=in_specs,
        out_specs=pl.BlockSpec((bm, d), lambda i: (i, 0)),
        out_shape=jax.ShapeDtypeStruct((n, d), jnp.float32),
        compiler_params=pltpu.CompilerParams(
            dimension_semantics=("arbitrary",),
        ),
    )(*([lap_matrix] * _SPLIT), features, features, W1.T, W2.T, bias)
